# baseline (speedup 1.0000x reference)
import functools

import jax
import jax.numpy as jnp
from jax import lax
from jax.experimental import pallas as pl
from jax.experimental.pallas import tpu as pltpu
from jax.experimental.pallas import tpu_sc as plsc

_K = 4608
_R = 8
_NBUF = 6
_BR = 256


def _sc_part(x, K):
    B, N, D = x.shape
    M = B * N
    xf = x.reshape(M, D)
    NW = 32
    rows_per_w = B * K // NW
    per_batch = NW // B
    n_chunks = rows_per_w // _R
    mesh = plsc.VectorSubcoreMesh(core_axis_name="c", subcore_axis_name="s")

    @functools.partial(
        pl.kernel,
        mesh=mesh,
        out_type=jax.ShapeDtypeStruct((M, D), jnp.float32),
        scratch_types=[
            pltpu.VMEM((rows_per_w,), jnp.int32),
            pltpu.VMEM((_NBUF, _R, D), jnp.float32),
        ]
        + [pltpu.SemaphoreType.DMA] * (2 * _NBUF),
    )
    def k(x_hbm, out_hbm, idx_flat, rows_v, *sems):
        gsems, wsems = sems[:_NBUF], sems[_NBUF:]
        wid = lax.axis_index("s") * 2 + lax.axis_index("c")
        batch = wid // per_batch
        base = batch * N + (wid % per_batch) * rows_per_w
        src0 = 2 * batch * N + N - 1 - base

        def fill(i, _):
            idx_flat[pl.ds(i * 16, 16)] = (
                jnp.full((16,), src0 - i * 16, jnp.int32) - lax.iota(jnp.int32, 16)
            )
            return 0

        lax.fori_loop(0, rows_per_w // 16, fill, 0)

        def start_gather(t, slot):
            pltpu.async_copy(
                x_hbm.at[idx_flat.at[pl.ds(t * _R, _R)]],
                rows_v.at[slot],
                gsems[slot],
            )

        def wait_gather(slot):
            pltpu.make_async_copy(
                x_hbm.at[idx_flat.at[pl.ds(0, _R)]], rows_v.at[slot], gsems[slot]
            ).wait()

        def start_write(t, slot):
            pltpu.async_copy(
                rows_v.at[slot], out_hbm.at[pl.ds(base + t * _R, _R)], wsems[slot]
            )

        def wait_write(slot):
            pltpu.make_async_copy(
                rows_v.at[slot], out_hbm.at[pl.ds(base, _R)], wsems[slot]
            ).wait()

        for s in range(_NBUF - 1):
            start_gather(s, s)

        def main_body(step, _):
            for u in range(_NBUF):
                t = step * _NBUF + u
                slot = u

                @pl.when(t < n_chunks)
                def _():
                    wait_gather(slot)
                    start_write(t, slot)
                    t2 = t + _NBUF - 1
                    slot2 = (u + _NBUF - 1) % _NBUF

                    @pl.when(t2 < n_chunks)
                    def _():
                        @pl.when(t2 >= _NBUF)
                        def _():
                            wait_write(slot2)

                        start_gather(t2, slot2)

            return 0

        nsteps = (n_chunks + _NBUF - 1) // _NBUF
        lax.fori_loop(0, nsteps, main_body, 0)
        for s in range(_NBUF):
            if any(t % _NBUF == s for t in range(max(0, n_chunks - _NBUF), n_chunks)):
                wait_write(s)

    return k(xf).reshape(B, N, D)


def _tc_body(x_ref, o_ref):
    v = x_ref[...]
    br = v.shape[1]
    g = jnp.concatenate(
        [v[:, br - 8 * (j + 1):br - 8 * j, :] for j in range(br // 8)], axis=1
    )
    i = lax.broadcasted_iota(jnp.int32, g.shape, 1)
    for s in (1, 2, 4):
        g = jnp.where((i & s) == 0, jnp.roll(g, -s, axis=1), jnp.roll(g, s, axis=1))
    o_ref[...] = g


def _tc_part(x, K):
    B, N, D = x.shape
    nb = (N - K) // _BR
    return pl.pallas_call(
        _tc_body,
        grid=(B, nb),
        in_specs=[pl.BlockSpec((1, _BR, D), lambda b, i: (b, nb - 1 - i, 0))],
        out_specs=pl.BlockSpec((1, _BR, D), lambda b, i: (b, i, 0)),
        out_shape=jax.ShapeDtypeStruct((B, N - K, D), x.dtype),
    )(x)


def kernel(x):
    sc_full = _sc_part(x, _K)
    tc_tail = _tc_part(x, _K)
    return lax.dynamic_update_slice(sc_full, tc_tail, (0, _K, 0))


# speedup vs baseline: 1.3005x; 1.3005x over previous
import functools

import jax
import jax.numpy as jnp
from jax import lax
from jax.experimental import pallas as pl
from jax.experimental.pallas import tpu as pltpu
from jax.experimental.pallas import tpu_sc as plsc

_R = 8
_NBUF = 6


def kernel(x):
    B, N, D = x.shape
    M = B * N
    xf = x.reshape(M, D)
    NW = 32
    rows_per_w = M // NW
    n_chunks = rows_per_w // _R
    mesh = plsc.VectorSubcoreMesh(core_axis_name="c", subcore_axis_name="s")

    @functools.partial(
        pl.kernel,
        mesh=mesh,
        out_type=jax.ShapeDtypeStruct((M, D), jnp.float32),
        scratch_types=[
            pltpu.VMEM((rows_per_w,), jnp.int32),
            pltpu.VMEM((_NBUF, _R, D), jnp.float32),
        ]
        + [pltpu.SemaphoreType.DMA] * (2 * _NBUF),
    )
    def k(x_hbm, out_hbm, idx_flat, rows_v, *sems):
        gsems, wsems = sems[:_NBUF], sems[_NBUF:]
        wid = lax.axis_index("s") * 2 + lax.axis_index("c")
        base = wid * rows_per_w
        src0 = 2 * (base // N) * N + N - 1 - base

        def fill(i, _):
            idx_flat[pl.ds(i * 16, 16)] = (
                jnp.full((16,), src0 - i * 16, jnp.int32) - lax.iota(jnp.int32, 16)
            )
            return 0

        lax.fori_loop(0, rows_per_w // 16, fill, 0)

        def buf(slot):
            return rows_v.at[slot]

        def start_gather(t, slot):
            pltpu.async_copy(
                x_hbm.at[idx_flat.at[pl.ds(t * _R, _R)]], buf(slot), gsems[slot]
            )

        def wait_gather(slot):
            pltpu.make_async_copy(
                x_hbm.at[idx_flat.at[pl.ds(0, _R)]], buf(slot), gsems[slot]
            ).wait()

        def start_write(t, slot):
            pltpu.async_copy(
                buf(slot), out_hbm.at[pl.ds(base + t * _R, _R)], wsems[slot]
            )

        def wait_write(slot):
            pltpu.make_async_copy(
                buf(slot), out_hbm.at[pl.ds(base, _R)], wsems[slot]
            ).wait()

        for s in range(_NBUF - 1):
            start_gather(s, s)

        def main_body(step, _):
            for u in range(_NBUF):
                t = step * _NBUF + u
                slot = u

                @pl.when(t < n_chunks)
                def _():
                    wait_gather(slot)
                    start_write(t, slot)
                    t2 = t + _NBUF - 1
                    slot2 = (u + _NBUF - 1) % _NBUF

                    @pl.when(t2 < n_chunks)
                    def _():
                        @pl.when(t2 >= _NBUF)
                        def _():
                            wait_write(slot2)

                        start_gather(t2, slot2)

            return 0

        nsteps = (n_chunks + _NBUF - 1) // _NBUF
        lax.fori_loop(0, nsteps, main_body, 0)
        for s in range(_NBUF):
            if any(t % _NBUF == s for t in range(max(0, n_chunks - _NBUF), n_chunks)):
                wait_write(s)

    return k(xf).reshape(B, N, D)
